# baseline (speedup 1.0000x reference)
import functools

import jax
import jax.numpy as jnp
from jax import lax
from jax.experimental import pallas as pl
from jax.experimental.pallas import tpu as pltpu
from jax.experimental.pallas import tpu_sc as plsc

NUM_CORES = 2
NUM_SUBCORES = 16
LANES = 16

BATCH = 16384
EMB = 16
BN_EPS = 1e-3


def _sc_body(bpw, ngroups, inv_std,
             idx2_hbm, utab_hbm, ubias_hbm, itab_hbm, ibias_hbm,
             gamma_hbm, beta_hbm, out_hbm,
             idx2_v, uidx_v, iidx_v, urows_v, irows_v, ubias_v, ibias_v,
             gamma_v, beta_v, out_v, sem):
    wid = lax.axis_index("s") * NUM_CORES + lax.axis_index("c")
    base = wid * bpw

    pltpu.sync_copy(idx2_hbm.at[pl.ds(2 * base, 2 * bpw)], idx2_v)
    pltpu.sync_copy(gamma_hbm, gamma_v)
    pltpu.sync_copy(beta_hbm, beta_v)

    lane = lax.iota(jnp.int32, LANES)

    def deinterleave(g, _):
        pos = (g * LANES + lane) * 2
        uidx_v[pl.ds(g * LANES, LANES)] = plsc.load_gather(idx2_v, [pos])
        iidx_v[pl.ds(g * LANES, LANES)] = plsc.load_gather(idx2_v, [pos + 1])
        return 0

    lax.fori_loop(0, ngroups, deinterleave, 0, unroll=4)

    cp_u = pltpu.make_async_copy(utab_hbm.at[uidx_v], urows_v, sem)
    cp_i = pltpu.make_async_copy(itab_hbm.at[iidx_v], irows_v, sem)
    cp_ub = pltpu.make_async_copy(ubias_hbm.at[uidx_v], ubias_v, sem)
    cp_ib = pltpu.make_async_copy(ibias_hbm.at[iidx_v], ibias_v, sem)
    cp_u.start()
    cp_i.start()
    cp_ub.start()
    cp_ib.start()
    cp_u.wait()
    cp_i.wait()
    cp_ub.wait()
    cp_ib.wait()

    scale = gamma_v[...] * inv_std
    beta_s = beta_v[...]

    def group(g, _):
        row = g * LANES + lane
        acc = ubias_v[pl.ds(g * LANES, LANES)] + ibias_v[pl.ds(g * LANES, LANES)]
        for d in range(EMB):
            col = jnp.full((LANES,), d, jnp.int32)
            uc = plsc.load_gather(urows_v, [row, col])
            ic = plsc.load_gather(irows_v, [row, col])
            acc = acc + uc * ic
        x = acc * scale + beta_s
        out_v[pl.ds(g * LANES, LANES)] = 1.0 / (1.0 + jnp.exp(-x))
        return 0

    lax.fori_loop(0, ngroups, group, 0, unroll=2)

    pltpu.sync_copy(out_v, out_hbm.at[pl.ds(base, bpw)])


def kernel(inputs, user_table, user_bias_table, item_table, item_bias_table,
           gamma, beta):
    batch = inputs.shape[0]
    nworkers = NUM_CORES * NUM_SUBCORES
    bpw = batch // nworkers
    ngroups = bpw // LANES
    inv_std = float(1.0 / (1.0 + BN_EPS) ** 0.5)

    idx2 = inputs.astype(jnp.int32).reshape(-1)
    ubias = user_bias_table.reshape(-1)
    ibias = item_bias_table.reshape(-1)

    mesh = plsc.VectorSubcoreMesh(
        core_axis_name="c", subcore_axis_name="s",
        num_cores=NUM_CORES, num_subcores=NUM_SUBCORES)

    run = pl.kernel(
        functools.partial(_sc_body, bpw, ngroups, inv_std),
        out_type=jax.ShapeDtypeStruct((batch,), jnp.float32),
        mesh=mesh,
        scratch_types=[
            pltpu.VMEM((2 * bpw,), jnp.int32),
            pltpu.VMEM((bpw,), jnp.int32),
            pltpu.VMEM((bpw,), jnp.int32),
            pltpu.VMEM((bpw, EMB), jnp.float32),
            pltpu.VMEM((bpw, EMB), jnp.float32),
            pltpu.VMEM((bpw,), jnp.float32),
            pltpu.VMEM((bpw,), jnp.float32),
            pltpu.VMEM((LANES,), jnp.float32),
            pltpu.VMEM((LANES,), jnp.float32),
            pltpu.VMEM((bpw,), jnp.float32),
            pltpu.SemaphoreType.DMA,
        ],
        compiler_params=pltpu.CompilerParams(
            needs_layout_passes=False, use_tc_tiling_on_sc=False),
    )
    gamma16 = jnp.broadcast_to(gamma.astype(jnp.float32).reshape(1), (LANES,))
    beta16 = jnp.broadcast_to(beta.astype(jnp.float32).reshape(1), (LANES,))
    out = run(idx2, user_table, ubias, item_table, ibias, gamma16, beta16)
    return out.reshape(batch, 1)
